# baseline (speedup 1.0000x reference)
import functools

import jax
import jax.numpy as jnp
from jax import lax
from jax.experimental import pallas as pl
from jax.experimental.pallas import tpu as pltpu
from jax.experimental.pallas import tpu_sc as plsc

BATCH = 16384
DIM = 64
NUM_CLASSES = 6

NC = 2
NS = 16
NW = NC * NS

IDX_CHUNK = 128
B_PER_W = BATCH // NW
NK = B_PER_W // IDX_CHUNK


def _sc_gather_body(idx_hbm, table_hbm, out_hbm, idx_v, rows_v, sem):
    wid = lax.axis_index("s") * NC + lax.axis_index("c")
    base = wid * B_PER_W
    pltpu.sync_copy(idx_hbm.at[pl.ds(wid * NK, NK)], idx_v)
    copies = [
        pltpu.async_copy(
            table_hbm.at[idx_v.at[j]],
            rows_v.at[pl.ds(j * IDX_CHUNK, IDX_CHUNK)],
            sem,
        )
        for j in range(NK)
    ]
    for c in copies:
        c.wait()
    pltpu.sync_copy(rows_v, out_hbm.at[pl.ds(base, B_PER_W)])


_sc_gather = pl.kernel(
    _sc_gather_body,
    out_type=jax.ShapeDtypeStruct((BATCH, DIM), jnp.float32),
    mesh=plsc.VectorSubcoreMesh(core_axis_name="c", subcore_axis_name="s"),
    scratch_types=[
        pltpu.VMEM((NK, IDX_CHUNK), jnp.int32),
        pltpu.VMEM((B_PER_W, DIM), jnp.float32),
        pltpu.SemaphoreType.DMA,
    ],
    compiler_params=pltpu.CompilerParams(use_tc_tiling_on_sc=False),
)


MM_BLOCK = 2048


def _mm_body(e_ref, w_ref, b_ref, o_ref):
    o_ref[...] = (
        jnp.dot(e_ref[...], w_ref[...], preferred_element_type=jnp.float32)
        + b_ref[...]
    )


_mm = pl.pallas_call(
    _mm_body,
    grid=(BATCH // MM_BLOCK,),
    in_specs=[
        pl.BlockSpec((MM_BLOCK, DIM), lambda i: (i, 0)),
        pl.BlockSpec((DIM, NUM_CLASSES), lambda i: (0, 0)),
        pl.BlockSpec((1, NUM_CLASSES), lambda i: (0, 0)),
    ],
    out_specs=pl.BlockSpec((MM_BLOCK, NUM_CLASSES), lambda i: (i, 0)),
    out_shape=jax.ShapeDtypeStruct((BATCH, NUM_CLASSES), jnp.float32),
)


@jax.jit
def kernel(idx, emb, W, b):
    idx2 = idx.astype(jnp.int32).reshape(BATCH // IDX_CHUNK, IDX_CHUNK)
    e = _sc_gather(idx2, emb)
    return _mm(e, W, b.reshape(1, NUM_CLASSES))
